# baseline (speedup 1.0000x reference)
import jax
import jax.numpy as jnp
from jax import lax
from jax.experimental import pallas as pl
from jax.experimental.pallas import tpu as pltpu
from jax.experimental.pallas import tpu_sc as plsc

SB = 0.1
N = 50000
PID_MAX = 1000

NUM_TILES = 16
LANES = 16
BINS = 1024
NOISE_COL = 1008
ITERS = -(-N // (NUM_TILES * LANES))
CHUNK = ITERS * LANES
LAST_BASE = N - CHUNK
OVERLAP_VECS = (NUM_TILES * CHUNK - N) // LANES


def _sc_body(beta_hbm, pid_hbm, out_hbm, beta_v, pid_v, bins, stg_row,
             pair_buf, stg, obuf, shstg, sem):
  wid = lax.axis_index("s")
  lane = lax.iota(jnp.int32, 16)
  zeros = jnp.zeros((16,), jnp.float32)
  ones = jnp.full((16,), 1.0, jnp.float32)
  minus1 = jnp.full((16,), -1.0, jnp.float32)

  base = jnp.where(wid == NUM_TILES - 1, LAST_BASE, wid * CHUNK)
  noise_start = jnp.where(wid == NUM_TILES - 1, OVERLAP_VECS, 0)
  cp_b = pltpu.async_copy(
      beta_hbm.at[pl.ds(base, CHUNK)], beta_v.at[pl.ds(0, CHUNK)], sem)
  cp_p = pltpu.async_copy(
      pid_hbm.at[pl.ds(base, CHUNK)], pid_v.at[pl.ds(0, CHUNK)], sem)

  def init_body(c, _):
    for k in range(LANES):
      bins[pl.ds(c * 256 + k * 16, 16)] = minus1
    return 0

  lax.fori_loop(0, LANES * BINS // 256, init_body, 0)
  cp_b.wait()
  cp_p.wait()

  lane_base = lane * BINS

  p0 = pid_v[pl.ds(0, 16)]
  b0v = beta_v[pl.ds(0, 16)]

  def scan_step(v, p, b, ns, nc):
    pn = pid_v[pl.ds(v * 16 + 16, 16)]
    bn = beta_v[pl.ds(v * 16 + 16, 16)]
    idx = lane_base + p
    old = plsc.load_gather(bins, [idx])
    plsc.store_scatter(bins, [idx], jnp.maximum(old, b))
    isn = (p == 0) & (v >= noise_start)
    ns = ns + jnp.where(isn, b, zeros)
    nc = nc + jnp.where(isn, ones, zeros)
    return pn, bn, ns, nc

  def scan_body(i, carry):
    p, b, ns, nc = carry
    p, b, ns, nc = scan_step(2 * i, p, b, ns, nc)
    p, b, ns, nc = scan_step(2 * i + 1, p, b, ns, nc)
    return p, b, ns, nc

  _, _, ns, nc = lax.fori_loop(
      0, ITERS // 2, scan_body, (p0, b0v, zeros, zeros))

  def lred_body(c, _):
    acc = bins[pl.ds(c * 16, 16)]
    for l in range(1, LANES):
      acc = jnp.maximum(acc, bins[pl.ds(l * BINS + c * 16, 16)])
    stg_row[pl.ds(c * 16, 16)] = acc
    return 0

  lax.fori_loop(0, BINS // 16, lred_body, 0)

  nsb = jnp.broadcast_to(jnp.sum(ns), (16,))
  ncb = jnp.broadcast_to(jnp.sum(nc), (16,))
  noise_pack = jnp.where(lane == 0, nsb, jnp.where(lane == 1, ncb, zeros))
  stg_row[pl.ds(NOISE_COL, 16)] = noise_pack
  pltpu.sync_copy(stg_row, shstg.at[wid])
  plsc.subcore_barrier()

  HALF = NUM_TILES // 2

  @pl.when(wid < HALF)
  def _pair():
    pltpu.sync_copy(shstg.at[wid + HALF], pair_buf)

    def pair_body(c, _):
      a = stg_row[pl.ds(c * 16, 16)]
      o = pair_buf[pl.ds(c * 16, 16)]
      stg_row[pl.ds(c * 16, 16)] = jnp.maximum(a, o)
      return 0

    lax.fori_loop(0, NOISE_COL // 16, pair_body, 0)
    stg_row[pl.ds(NOISE_COL, 16)] = (
        stg_row[pl.ds(NOISE_COL, 16)] + pair_buf[pl.ds(NOISE_COL, 16)])
    pltpu.sync_copy(stg_row, shstg.at[wid])

  plsc.subcore_barrier()

  @pl.when(wid == 0)
  def _final():
    pltpu.sync_copy(shstg.at[pl.ds(0, HALF)], stg)
    b0 = beta_v[pl.ds(0, 16)][0]

    def fin_body(c, carry):
      st, pc = carry
      m = stg[0, pl.ds(c * 16, 16)]
      for l in range(1, HALF):
        m = jnp.maximum(m, stg[l, pl.ds(c * 16, 16)])
      cols = c * 16 + lane
      present = (cols >= 1) & (cols < PID_MAX) & (m >= 0.0)
      ba = jnp.where(m > 0.0, m, b0)
      st = st + jnp.where(present, 1.0 - ba, zeros)
      pc = pc + jnp.where(present, ones, zeros)
      return st, pc

    st, pc = lax.fori_loop(0, BINS // 16, fin_body, (zeros, zeros))
    nacc = zeros
    for r in range(HALF):
      nacc = nacc + stg[r, pl.ds(NOISE_COL, 16)]
    sum_terms = jnp.broadcast_to(jnp.sum(st), (16,))
    pcnt = jnp.broadcast_to(jnp.sum(pc), (16,))
    nsum = jnp.broadcast_to(nacc[0], (16,))
    ncnt = jnp.broadcast_to(nacc[1], (16,))
    def refined_div(x, y):
      q = x / y
      return q + (x - q * y) / y

    loss = refined_div(sum_terms, pcnt)
    loss = jnp.where(ncnt > 0.0, loss + SB * refined_div(nsum, ncnt), loss)
    obuf[...] = loss
    pltpu.sync_copy(obuf, out_hbm)


@jax.jit
def _sc_loss(beta_p, pid_p):
  mesh = plsc.VectorSubcoreMesh(
      core_axis_name="c", subcore_axis_name="s", num_cores=1)
  f = pl.kernel(
      _sc_body,
      out_type=jax.ShapeDtypeStruct((16,), jnp.float32),
      mesh=mesh,
      scratch_types=[
          pltpu.VMEM((CHUNK + 16,), jnp.float32),
          pltpu.VMEM((CHUNK + 16,), jnp.int32),
          pltpu.VMEM((LANES * BINS,), jnp.float32),
          pltpu.VMEM((BINS,), jnp.float32),
          pltpu.VMEM((BINS,), jnp.float32),
          pltpu.VMEM((NUM_TILES // 2, BINS), jnp.float32),
          pltpu.VMEM((16,), jnp.float32),
          pltpu.VMEM_SHARED((NUM_TILES, BINS), jnp.float32),
          pltpu.SemaphoreType.DMA,
      ],
      compiler_params=pltpu.CompilerParams(needs_layout_passes=False),
  )
  return f(beta_p, pid_p)


def kernel(beta, particle_id):
  out = _sc_loss(beta, particle_id)
  return out[0]


# speedup vs baseline: 1.0031x; 1.0031x over previous
import jax
import jax.numpy as jnp
from jax import lax
from jax.experimental import pallas as pl
from jax.experimental.pallas import tpu as pltpu
from jax.experimental.pallas import tpu_sc as plsc

SB = 0.1
N = 50000
PID_MAX = 1000

NUM_TILES = 16
LANES = 16
BINS = 1024
NOISE_COL = 1008
ITERS = -(-N // (NUM_TILES * LANES))
CHUNK = ITERS * LANES
LAST_BASE = N - CHUNK
OVERLAP_VECS = (NUM_TILES * CHUNK - N) // LANES


def _sc_body(beta_hbm, pid_hbm, out_hbm, beta_v, pid_v, bins, stg_row,
             stg, obuf, shstg, sem):
  wid = lax.axis_index("s")
  lane = lax.iota(jnp.int32, 16)
  zeros = jnp.zeros((16,), jnp.float32)
  ones = jnp.full((16,), 1.0, jnp.float32)
  minus1 = jnp.full((16,), -1.0, jnp.float32)

  base = jnp.where(wid == NUM_TILES - 1, LAST_BASE, wid * CHUNK)
  noise_start = jnp.where(wid == NUM_TILES - 1, OVERLAP_VECS, 0)
  cp_b = pltpu.async_copy(
      beta_hbm.at[pl.ds(base, CHUNK)], beta_v.at[pl.ds(0, CHUNK)], sem)
  cp_p = pltpu.async_copy(
      pid_hbm.at[pl.ds(base, CHUNK)], pid_v.at[pl.ds(0, CHUNK)], sem)

  def init_body(c, _):
    for k in range(LANES):
      bins[pl.ds(c * 256 + k * 16, 16)] = minus1
    return 0

  lax.fori_loop(0, LANES * BINS // 256, init_body, 0)
  cp_b.wait()
  cp_p.wait()

  lane_base = lane * BINS

  p0 = pid_v[pl.ds(0, 16)]
  b0v = beta_v[pl.ds(0, 16)]

  def scan_step(v, p, b, ns, nc):
    pn = pid_v[pl.ds(v * 16 + 16, 16)]
    bn = beta_v[pl.ds(v * 16 + 16, 16)]
    idx = lane_base + p
    old = plsc.load_gather(bins, [idx])
    plsc.store_scatter(bins, [idx], jnp.maximum(old, b))
    isn = (p == 0) & (v >= noise_start)
    ns = ns + jnp.where(isn, b, zeros)
    nc = nc + jnp.where(isn, ones, zeros)
    return pn, bn, ns, nc

  def scan_body(i, carry):
    p, b, ns, nc = carry
    p, b, ns, nc = scan_step(2 * i, p, b, ns, nc)
    p, b, ns, nc = scan_step(2 * i + 1, p, b, ns, nc)
    return p, b, ns, nc

  _, _, ns, nc = lax.fori_loop(
      0, ITERS // 2, scan_body, (p0, b0v, zeros, zeros))

  def lred_body(c, _):
    acc = bins[pl.ds(c * 16, 16)]
    for l in range(1, LANES):
      acc = jnp.maximum(acc, bins[pl.ds(l * BINS + c * 16, 16)])
    stg_row[pl.ds(c * 16, 16)] = acc
    return 0

  lax.fori_loop(0, BINS // 16, lred_body, 0)

  nsb = jnp.broadcast_to(jnp.sum(ns), (16,))
  ncb = jnp.broadcast_to(jnp.sum(nc), (16,))
  noise_pack = jnp.where(lane == 0, nsb, jnp.where(lane == 1, ncb, zeros))
  stg_row[pl.ds(NOISE_COL, 16)] = noise_pack
  pltpu.sync_copy(stg_row, shstg.at[wid])
  plsc.subcore_barrier()

  @pl.when(wid == 0)
  def _final():
    pltpu.sync_copy(shstg, stg)
    b0 = beta_v[pl.ds(0, 16)][0]

    def fin_body(c, carry):
      st, pc = carry
      m = stg[0, pl.ds(c * 16, 16)]
      for l in range(1, NUM_TILES):
        m = jnp.maximum(m, stg[l, pl.ds(c * 16, 16)])
      cols = c * 16 + lane
      present = (cols >= 1) & (cols < PID_MAX) & (m >= 0.0)
      ba = jnp.where(m > 0.0, m, b0)
      st = st + jnp.where(present, 1.0 - ba, zeros)
      pc = pc + jnp.where(present, ones, zeros)
      return st, pc

    st, pc = lax.fori_loop(0, BINS // 16, fin_body, (zeros, zeros))
    nacc = zeros
    for r in range(NUM_TILES):
      nacc = nacc + stg[r, pl.ds(NOISE_COL, 16)]
    sum_terms = jnp.broadcast_to(jnp.sum(st), (16,))
    pcnt = jnp.broadcast_to(jnp.sum(pc), (16,))
    nsum = jnp.broadcast_to(nacc[0], (16,))
    ncnt = jnp.broadcast_to(nacc[1], (16,))
    def refined_div(x, y):
      q = x / y
      return q + (x - q * y) / y

    loss = refined_div(sum_terms, pcnt)
    loss = jnp.where(ncnt > 0.0, loss + SB * refined_div(nsum, ncnt), loss)
    obuf[...] = loss
    pltpu.sync_copy(obuf, out_hbm)


@jax.jit
def _sc_loss(beta_p, pid_p):
  mesh = plsc.VectorSubcoreMesh(
      core_axis_name="c", subcore_axis_name="s", num_cores=1)
  f = pl.kernel(
      _sc_body,
      out_type=jax.ShapeDtypeStruct((16,), jnp.float32),
      mesh=mesh,
      scratch_types=[
          pltpu.VMEM((CHUNK + 16,), jnp.float32),
          pltpu.VMEM((CHUNK + 16,), jnp.int32),
          pltpu.VMEM((LANES * BINS,), jnp.float32),
          pltpu.VMEM((BINS,), jnp.float32),
          pltpu.VMEM((NUM_TILES, BINS), jnp.float32),
          pltpu.VMEM((16,), jnp.float32),
          pltpu.VMEM_SHARED((NUM_TILES, BINS), jnp.float32),
          pltpu.SemaphoreType.DMA,
      ],
      compiler_params=pltpu.CompilerParams(needs_layout_passes=False),
  )
  return f(beta_p, pid_p)


def kernel(beta, particle_id):
  out = _sc_loss(beta, particle_id)
  return out[0]


# speedup vs baseline: 1.0156x; 1.0125x over previous
import jax
import jax.numpy as jnp
from jax import lax
from jax.experimental import pallas as pl
from jax.experimental.pallas import tpu as pltpu
from jax.experimental.pallas import tpu_sc as plsc

SB = 0.1
N = 50000
PID_MAX = 1000

NUM_TILES = 16
LANES = 16
BINS = 1024
NOISE_COL = 1008
ITERS = -(-N // (NUM_TILES * LANES))
CHUNK = ITERS * LANES
LAST_BASE = N - CHUNK
OVERLAP_VECS = (NUM_TILES * CHUNK - N) // LANES


def _sc_body(beta_hbm, pid_hbm, out_hbm, beta_v, pid_v, bins, stg_row,
             stg, obuf, shstg, sem):
  wid = lax.axis_index("s")
  lane = lax.iota(jnp.int32, 16)
  zeros = jnp.zeros((16,), jnp.float32)
  ones = jnp.full((16,), 1.0, jnp.float32)
  minus1 = jnp.full((16,), -1.0, jnp.float32)

  base = jnp.where(wid == NUM_TILES - 1, LAST_BASE, wid * CHUNK)
  noise_start = jnp.where(wid == NUM_TILES - 1, OVERLAP_VECS, 0)
  cp_b = pltpu.async_copy(
      beta_hbm.at[pl.ds(base, CHUNK)], beta_v.at[pl.ds(0, CHUNK)], sem)
  cp_p = pltpu.async_copy(
      pid_hbm.at[pl.ds(base, CHUNK)], pid_v.at[pl.ds(0, CHUNK)], sem)

  def init_body(c, _):
    for k in range(LANES):
      bins[pl.ds(c * 256 + k * 16, 16)] = minus1
    return 0

  lax.fori_loop(0, LANES * BINS // 256, init_body, 0)
  cp_b.wait()
  cp_p.wait()

  lane_base = lane * BINS

  p0 = pid_v[pl.ds(0, 16)]
  b0v = beta_v[pl.ds(0, 16)]

  def scan_body(i, carry):
    p, b, ns, nc = carry
    pn = pid_v[pl.ds(i * 16 + 16, 16)]
    bn = beta_v[pl.ds(i * 16 + 16, 16)]
    idx = lane_base + p
    old = plsc.load_gather(bins, [idx])
    plsc.store_scatter(bins, [idx], jnp.maximum(old, b))
    isn = (p == 0) & (i >= noise_start)
    ns = ns + jnp.where(isn, b, zeros)
    nc = nc + jnp.where(isn, ones, zeros)
    return pn, bn, ns, nc

  _, _, ns, nc = lax.fori_loop(0, ITERS, scan_body, (p0, b0v, zeros, zeros))

  def lred_body(c, _):
    acc = bins[pl.ds(c * 16, 16)]
    for l in range(1, LANES):
      acc = jnp.maximum(acc, bins[pl.ds(l * BINS + c * 16, 16)])
    stg_row[pl.ds(c * 16, 16)] = acc
    return 0

  lax.fori_loop(0, BINS // 16, lred_body, 0)

  nsb = jnp.broadcast_to(jnp.sum(ns), (16,))
  ncb = jnp.broadcast_to(jnp.sum(nc), (16,))
  noise_pack = jnp.where(lane == 0, nsb, jnp.where(lane == 1, ncb, zeros))
  stg_row[pl.ds(NOISE_COL, 16)] = noise_pack
  pltpu.sync_copy(stg_row, shstg.at[wid])
  plsc.subcore_barrier()

  @pl.when(wid == 0)
  def _final():
    pltpu.sync_copy(shstg, stg)
    b0 = beta_v[pl.ds(0, 16)][0]

    def fin_body(c, carry):
      st, pc = carry
      m = stg[0, pl.ds(c * 16, 16)]
      for l in range(1, NUM_TILES):
        m = jnp.maximum(m, stg[l, pl.ds(c * 16, 16)])
      cols = c * 16 + lane
      present = (cols >= 1) & (cols < PID_MAX) & (m >= 0.0)
      ba = jnp.where(m > 0.0, m, b0)
      st = st + jnp.where(present, 1.0 - ba, zeros)
      pc = pc + jnp.where(present, ones, zeros)
      return st, pc

    st, pc = lax.fori_loop(0, BINS // 16, fin_body, (zeros, zeros))
    nacc = zeros
    for r in range(NUM_TILES):
      nacc = nacc + stg[r, pl.ds(NOISE_COL, 16)]
    sum_terms = jnp.broadcast_to(jnp.sum(st), (16,))
    pcnt = jnp.broadcast_to(jnp.sum(pc), (16,))
    nsum = jnp.broadcast_to(nacc[0], (16,))
    ncnt = jnp.broadcast_to(nacc[1], (16,))
    def refined_div(x, y):
      q = x / y
      return q + (x - q * y) / y

    loss = refined_div(sum_terms, pcnt)
    loss = jnp.where(ncnt > 0.0, loss + SB * refined_div(nsum, ncnt), loss)
    obuf[...] = loss
    pltpu.sync_copy(obuf, out_hbm)


@jax.jit
def _sc_loss(beta_p, pid_p):
  mesh = plsc.VectorSubcoreMesh(
      core_axis_name="c", subcore_axis_name="s", num_cores=1)
  f = pl.kernel(
      _sc_body,
      out_type=jax.ShapeDtypeStruct((16,), jnp.float32),
      mesh=mesh,
      scratch_types=[
          pltpu.VMEM((CHUNK + 16,), jnp.float32),
          pltpu.VMEM((CHUNK + 16,), jnp.int32),
          pltpu.VMEM((LANES * BINS,), jnp.float32),
          pltpu.VMEM((BINS,), jnp.float32),
          pltpu.VMEM((NUM_TILES, BINS), jnp.float32),
          pltpu.VMEM((16,), jnp.float32),
          pltpu.VMEM_SHARED((NUM_TILES, BINS), jnp.float32),
          pltpu.SemaphoreType.DMA,
      ],
      compiler_params=pltpu.CompilerParams(needs_layout_passes=False),
  )
  return f(beta_p, pid_p)


def kernel(beta, particle_id):
  out = _sc_loss(beta, particle_id)
  return out[0]
